# baseline (speedup 1.0000x reference)
import functools

import jax
import jax.numpy as jnp
from jax import lax
from jax.experimental import pallas as pl
from jax.experimental.pallas import tpu as pltpu
from jax.experimental.pallas import tpu_sc as plsc

K = 3


def _topk_body(p_ref, st_ref, i0_ref, i1_ref, i2_ref, w0_ref, w1_ref, w2_ref,
               *, ns, b0):
    tn = p_ref.shape[0]
    p = p_ref[...]
    st = st_ref[0]
    pp = jnp.sum(p * p, axis=1, keepdims=True)
    ss = jnp.sum(st * st, axis=0, keepdims=True)
    dot = jax.lax.dot_general(
        p, st, dimension_numbers=(((1,), (0,)), ((), ())),
        preferred_element_type=jnp.float32)
    d2 = pp + ss - 2.0 * dot

    iota = lax.broadcasted_iota(jnp.int32, (tn, ns), 1)
    vals = []
    idxs = []
    d = d2
    for k in range(K):
        vmin = jnp.min(d, axis=1, keepdims=True)
        sel = jnp.where(d == vmin, iota, ns)
        ik = jnp.min(sel, axis=1, keepdims=True)
        if k < K - 1:
            d = jnp.where(iota == ik, jnp.float32(3e38), d)
        vals.append(vmin)
        idxs.append(ik)

    r0 = 1.0 / (jnp.sqrt(jnp.maximum(vals[0], 0.0)) + 1e-8)
    r1 = 1.0 / (jnp.sqrt(jnp.maximum(vals[1], 0.0)) + 1e-8)
    r2 = 1.0 / (jnp.sqrt(jnp.maximum(vals[2], 0.0)) + 1e-8)
    norm = r0 + r1 + r2
    w0_ref[...] = r0 / norm
    w1_ref[...] = r1 / norm
    w2_ref[...] = r2 / norm
    boff = (pl.program_id(0) + b0) * ns
    i0_ref[...] = idxs[0] + boff
    i1_ref[...] = idxs[1] + boff
    i2_ref[...] = idxs[2] + boff


def _topk_weights(parent_coord, s_coord_t, *, b, nn, ns, tn, b0=0):
    nt = nn // tn
    grid = (b, nt)
    spec1 = pl.BlockSpec((tn, 1), lambda bi, i: (bi * (nn // tn) + i, 0))
    return pl.pallas_call(
        functools.partial(_topk_body, ns=ns, b0=b0),
        grid=grid,
        in_specs=[
            pl.BlockSpec((tn, 3), lambda bi, i: (bi * (nn // tn) + i, 0)),
            pl.BlockSpec((1, 3, ns), lambda bi, i: (bi, 0, 0)),
        ],
        out_specs=[spec1] * 6,
        out_shape=[jax.ShapeDtypeStruct((b * nn, 1), jnp.int32)] * 3
        + [jax.ShapeDtypeStruct((b * nn, 1), jnp.float32)] * 3,
    )(parent_coord, s_coord_t)


def _make_sc_gather(bnn, d, c):
    nc, nsub = 2, 16
    nw = nc * nsub
    ppw = bnn // nw
    nchunk = ppw // c
    mesh = plsc.VectorSubcoreMesh(
        core_axis_name="c", subcore_axis_name="s", num_cores=nc)

    @functools.partial(
        pl.kernel,
        out_type=jax.ShapeDtypeStruct((bnn, d), jnp.float32),
        mesh=mesh,
        scratch_types=[
            pltpu.VMEM((c,), jnp.int32),
            pltpu.VMEM((c,), jnp.int32),
            pltpu.VMEM((c,), jnp.int32),
            pltpu.VMEM((c,), jnp.float32),
            pltpu.VMEM((c,), jnp.float32),
            pltpu.VMEM((c,), jnp.float32),
            pltpu.VMEM((c, d), jnp.float32),
            pltpu.VMEM((c, d), jnp.float32),
            pltpu.VMEM((c, d), jnp.float32),
            pltpu.SemaphoreType.DMA,
        ],
    )
    def sc_gather(idx0_hbm, idx1_hbm, idx2_hbm, w0_hbm, w1_hbm, w2_hbm,
                  feat_hbm, out_hbm,
                  i0_v, i1_v, i2_v, w0_v, w1_v, w2_v, r0, r1, r2, sem):
        wid = lax.axis_index("s") * nc + lax.axis_index("c")

        def chunk_body(ci, carry):
            base = wid * ppw + ci * c
            sl = pl.ds(base, c)
            pltpu.sync_copy(idx0_hbm.at[sl], i0_v)
            pltpu.sync_copy(idx1_hbm.at[sl], i1_v)
            pltpu.sync_copy(idx2_hbm.at[sl], i2_v)
            pltpu.sync_copy(w0_hbm.at[sl], w0_v)
            pltpu.sync_copy(w1_hbm.at[sl], w1_v)
            pltpu.sync_copy(w2_hbm.at[sl], w2_v)
            cp0 = pltpu.async_copy(feat_hbm.at[i0_v], r0, sem)
            cp1 = pltpu.async_copy(feat_hbm.at[i1_v], r1, sem)
            cp2 = pltpu.async_copy(feat_hbm.at[i2_v], r2, sem)
            cp0.wait()
            cp1.wait()
            cp2.wait()

            def group_body(g, pcarry):
                g16 = g * 16
                w0g = w0_v[pl.ds(g16, 16)]
                w1g = w1_v[pl.ds(g16, 16)]
                w2g = w2_v[pl.ds(g16, 16)]
                for j in range(16):
                    w0, w1, w2 = w0g[j], w1g[j], w2g[j]
                    pi = g16 + j
                    for db in range(d // 16):
                        ds = pl.ds(db * 16, 16)
                        r0[pi, ds] = (w0 * r0[pi, ds] + w1 * r1[pi, ds]
                                      + w2 * r2[pi, ds])
                return pcarry

            lax.fori_loop(0, c // 16, group_body, 0)
            pltpu.sync_copy(r0, out_hbm.at[sl])
            return carry

        lax.fori_loop(0, nchunk, chunk_body, 0)

    return sc_gather


def kernel(parent_coord, s_coord, s_feat, offset, new_offset):
    b = offset.shape[0]
    ns = s_coord.shape[0] // b
    nn = parent_coord.shape[0] // b
    d = s_feat.shape[1]
    tn = 256

    s_coord_t = s_coord.reshape(b, ns, 3).transpose(0, 2, 1)

    bs = 2
    sc = _make_sc_gather(bs * nn, d, 128)
    outs = []
    for g in range(0, b, bs):
        i0, i1, i2, w0, w1, w2 = _topk_weights(
            parent_coord[g * nn:(g + bs) * nn], s_coord_t[g:g + bs],
            b=bs, nn=nn, ns=ns, tn=tn, b0=g)
        outs.append(sc(i0.reshape(-1), i1.reshape(-1), i2.reshape(-1),
                       w0.reshape(-1), w1.reshape(-1), w2.reshape(-1),
                       s_feat))
    return jnp.concatenate(outs, axis=0)


# speedup vs baseline: 1.1917x; 1.1917x over previous
import functools

import jax
import jax.numpy as jnp
from jax import lax
from jax.experimental import pallas as pl
from jax.experimental.pallas import tpu as pltpu
from jax.experimental.pallas import tpu_sc as plsc

K = 3


def _topk_body(pt_ref, s_ref, i0_ref, i1_ref, i2_ref, w0_ref, w1_ref, w2_ref,
               *, ns, b0):
    tn = pt_ref.shape[1]
    pt = pt_ref[...]
    s = s_ref[...]
    pp = jnp.sum(pt * pt, axis=0, keepdims=True)
    ss = jnp.sum(s * s, axis=1, keepdims=True)
    dot = jax.lax.dot_general(
        s, pt, dimension_numbers=(((1,), (0,)), ((), ())),
        preferred_element_type=jnp.float32)
    d2 = pp + ss - 2.0 * dot

    iota = lax.broadcasted_iota(jnp.int32, (ns, tn), 0)
    vals = []
    idxs = []
    d = d2
    for k in range(K):
        vmin = jnp.min(d, axis=0, keepdims=True)
        sel = jnp.where(d == vmin, iota, ns)
        ik = jnp.min(sel, axis=0, keepdims=True)
        if k < K - 1:
            d = jnp.where(iota == ik, jnp.float32(3e38), d)
        vals.append(vmin)
        idxs.append(ik)

    r0 = 1.0 / (jnp.sqrt(jnp.maximum(vals[0], 0.0)) + 1e-8)
    r1 = 1.0 / (jnp.sqrt(jnp.maximum(vals[1], 0.0)) + 1e-8)
    r2 = 1.0 / (jnp.sqrt(jnp.maximum(vals[2], 0.0)) + 1e-8)
    norm = r0 + r1 + r2
    one = (1, 1, tn)
    w0_ref[...] = (r0 / norm).reshape(one)
    w1_ref[...] = (r1 / norm).reshape(one)
    w2_ref[...] = (r2 / norm).reshape(one)
    boff = (pl.program_id(0) + b0) * ns
    i0_ref[...] = (idxs[0] + boff).reshape(one)
    i1_ref[...] = (idxs[1] + boff).reshape(one)
    i2_ref[...] = (idxs[2] + boff).reshape(one)


def _topk_weights(parent_coord_t, s_coord, *, b, nn, ns, tn, b0=0):
    nt = nn // tn
    grid = (b, nt)
    spec1 = pl.BlockSpec((1, 1, tn), lambda bi, i: (bi * (nn // tn) + i, 0, 0))
    return pl.pallas_call(
        functools.partial(_topk_body, ns=ns, b0=b0),
        grid=grid,
        in_specs=[
            pl.BlockSpec((3, tn), lambda bi, i: (0, bi * (nn // tn) + i)),
            pl.BlockSpec((ns, 3), lambda bi, i: (bi, 0)),
        ],
        out_specs=[spec1] * 6,
        out_shape=[jax.ShapeDtypeStruct((b * nn // tn, 1, tn), jnp.int32)] * 3
        + [jax.ShapeDtypeStruct((b * nn // tn, 1, tn), jnp.float32)] * 3,
    )(parent_coord_t, s_coord)


def _make_sc_gather(bnn, d, c):
    nc, nsub = 2, 16
    nw = nc * nsub
    ppw = bnn // nw
    nchunk = ppw // c
    mesh = plsc.VectorSubcoreMesh(
        core_axis_name="c", subcore_axis_name="s", num_cores=nc)

    @functools.partial(
        pl.kernel,
        out_type=jax.ShapeDtypeStruct((bnn, d), jnp.float32),
        mesh=mesh,
        scratch_types=[
            pltpu.VMEM((c,), jnp.int32),
            pltpu.VMEM((c,), jnp.int32),
            pltpu.VMEM((c,), jnp.int32),
            pltpu.VMEM((c,), jnp.float32),
            pltpu.VMEM((c,), jnp.float32),
            pltpu.VMEM((c,), jnp.float32),
            pltpu.VMEM((c, d), jnp.float32),
            pltpu.VMEM((c, d), jnp.float32),
            pltpu.VMEM((c, d), jnp.float32),
            pltpu.SemaphoreType.DMA,
        ],
    )
    def sc_gather(idx0_hbm, idx1_hbm, idx2_hbm, w0_hbm, w1_hbm, w2_hbm,
                  feat_hbm, out_hbm,
                  i0_v, i1_v, i2_v, w0_v, w1_v, w2_v, r0, r1, r2, sem):
        wid = lax.axis_index("s") * nc + lax.axis_index("c")

        def chunk_body(ci, carry):
            base = wid * ppw + ci * c
            sl = pl.ds(base, c)
            pltpu.sync_copy(idx0_hbm.at[sl], i0_v)
            pltpu.sync_copy(idx1_hbm.at[sl], i1_v)
            pltpu.sync_copy(idx2_hbm.at[sl], i2_v)
            pltpu.sync_copy(w0_hbm.at[sl], w0_v)
            pltpu.sync_copy(w1_hbm.at[sl], w1_v)
            pltpu.sync_copy(w2_hbm.at[sl], w2_v)
            cp0 = pltpu.async_copy(feat_hbm.at[i0_v], r0, sem)
            cp1 = pltpu.async_copy(feat_hbm.at[i1_v], r1, sem)
            cp2 = pltpu.async_copy(feat_hbm.at[i2_v], r2, sem)
            cp0.wait()
            cp1.wait()
            cp2.wait()

            def group_body(g, pcarry):
                g16 = g * 16
                w0g = w0_v[pl.ds(g16, 16)]
                w1g = w1_v[pl.ds(g16, 16)]
                w2g = w2_v[pl.ds(g16, 16)]
                for j in range(16):
                    w0, w1, w2 = w0g[j], w1g[j], w2g[j]
                    pi = g16 + j
                    for db in range(d // 16):
                        ds = pl.ds(db * 16, 16)
                        r0[pi, ds] = (w0 * r0[pi, ds] + w1 * r1[pi, ds]
                                      + w2 * r2[pi, ds])
                return pcarry

            lax.fori_loop(0, c // 16, group_body, 0)
            pltpu.sync_copy(r0, out_hbm.at[sl])
            return carry

        lax.fori_loop(0, nchunk, chunk_body, 0)

    return sc_gather


def kernel(parent_coord, s_coord, s_feat, offset, new_offset):
    b = offset.shape[0]
    ns = s_coord.shape[0] // b
    nn = parent_coord.shape[0] // b
    d = s_feat.shape[1]
    tn = 256

    parent_coord_t = parent_coord.T

    bs = 2
    sc = _make_sc_gather(bs * nn, d, 128)
    outs = []
    for g in range(0, b, bs):
        i0, i1, i2, w0, w1, w2 = _topk_weights(
            parent_coord_t[:, g * nn:(g + bs) * nn],
            s_coord[g * ns:(g + bs) * ns],
            b=bs, nn=nn, ns=ns, tn=tn, b0=g)
        outs.append(sc(i0.reshape(-1), i1.reshape(-1), i2.reshape(-1),
                       w0.reshape(-1), w1.reshape(-1), w2.reshape(-1),
                       s_feat))
    return jnp.concatenate(outs, axis=0)


# speedup vs baseline: 1.2744x; 1.0694x over previous
import functools

import jax
import jax.numpy as jnp
from jax import lax
from jax.experimental import pallas as pl
from jax.experimental.pallas import tpu as pltpu
from jax.experimental.pallas import tpu_sc as plsc

K = 3


def _topk_body(pt_ref, s_ref, i0_ref, i1_ref, i2_ref, w0_ref, w1_ref, w2_ref,
               *, ns, b0):
    tn = pt_ref.shape[1]
    pt = pt_ref[...]
    s = s_ref[...]
    pp = jnp.sum(pt * pt, axis=0, keepdims=True)
    ss = jnp.sum(s * s, axis=1, keepdims=True)
    dot = jax.lax.dot_general(
        s, pt, dimension_numbers=(((1,), (0,)), ((), ())),
        preferred_element_type=jnp.float32)
    d2 = pp + ss - 2.0 * dot

    iota = lax.broadcasted_iota(jnp.int32, (ns, tn), 0)
    vals = []
    idxs = []
    d = d2
    for k in range(K):
        vmin = jnp.min(d, axis=0, keepdims=True)
        sel = jnp.where(d == vmin, iota, ns)
        ik = jnp.min(sel, axis=0, keepdims=True)
        if k < K - 1:
            d = jnp.where(iota == ik, jnp.float32(3e38), d)
        vals.append(vmin)
        idxs.append(ik)

    r0 = 1.0 / (jnp.sqrt(jnp.maximum(vals[0], 0.0)) + 1e-8)
    r1 = 1.0 / (jnp.sqrt(jnp.maximum(vals[1], 0.0)) + 1e-8)
    r2 = 1.0 / (jnp.sqrt(jnp.maximum(vals[2], 0.0)) + 1e-8)
    norm = r0 + r1 + r2
    one = (1, 1, tn)
    w0_ref[...] = (r0 / norm).reshape(one)
    w1_ref[...] = (r1 / norm).reshape(one)
    w2_ref[...] = (r2 / norm).reshape(one)
    boff = (pl.program_id(0) + b0) * ns
    i0_ref[...] = (idxs[0] + boff).reshape(one)
    i1_ref[...] = (idxs[1] + boff).reshape(one)
    i2_ref[...] = (idxs[2] + boff).reshape(one)


def _topk_weights(parent_coord_t, s_coord, *, b, nn, ns, tn, b0=0):
    nt = nn // tn
    grid = (b, nt)
    spec1 = pl.BlockSpec((1, 1, tn), lambda bi, i: (bi * (nn // tn) + i, 0, 0))
    return pl.pallas_call(
        functools.partial(_topk_body, ns=ns, b0=b0),
        grid=grid,
        in_specs=[
            pl.BlockSpec((3, tn), lambda bi, i: (0, bi * (nn // tn) + i)),
            pl.BlockSpec((ns, 3), lambda bi, i: (bi, 0)),
        ],
        out_specs=[spec1] * 6,
        out_shape=[jax.ShapeDtypeStruct((b * nn // tn, 1, tn), jnp.int32)] * 3
        + [jax.ShapeDtypeStruct((b * nn // tn, 1, tn), jnp.float32)] * 3,
    )(parent_coord_t, s_coord)


def _make_sc_gather(bnn, d, c):
    nc, nsub = 2, 16
    nw = nc * nsub
    ppw = bnn // nw
    nchunk = ppw // c
    mesh = plsc.VectorSubcoreMesh(
        core_axis_name="c", subcore_axis_name="s", num_cores=nc)

    @functools.partial(
        pl.kernel,
        out_type=jax.ShapeDtypeStruct((bnn, d), jnp.float32),
        mesh=mesh,
        scratch_types=[
            pltpu.VMEM((c,), jnp.int32),
            pltpu.VMEM((c,), jnp.int32),
            pltpu.VMEM((c,), jnp.int32),
            pltpu.VMEM((c,), jnp.float32),
            pltpu.VMEM((c,), jnp.float32),
            pltpu.VMEM((c,), jnp.float32),
            pltpu.VMEM((c, d), jnp.float32),
            pltpu.VMEM((c, d), jnp.float32),
            pltpu.VMEM((c, d), jnp.float32),
            pltpu.SemaphoreType.DMA,
        ],
    )
    def sc_gather(idx0_hbm, idx1_hbm, idx2_hbm, w0_hbm, w1_hbm, w2_hbm,
                  feat_hbm, out_hbm,
                  i0_v, i1_v, i2_v, w0_v, w1_v, w2_v, r0, r1, r2, sem):
        wid = lax.axis_index("s") * nc + lax.axis_index("c")

        def chunk_body(ci, carry):
            base = wid * ppw + ci * c
            sl = pl.ds(base, c)
            pltpu.sync_copy(idx0_hbm.at[sl], i0_v)
            pltpu.sync_copy(idx1_hbm.at[sl], i1_v)
            pltpu.sync_copy(idx2_hbm.at[sl], i2_v)
            pltpu.sync_copy(w0_hbm.at[sl], w0_v)
            pltpu.sync_copy(w1_hbm.at[sl], w1_v)
            pltpu.sync_copy(w2_hbm.at[sl], w2_v)
            cp0 = pltpu.async_copy(feat_hbm.at[i0_v], r0, sem)
            cp1 = pltpu.async_copy(feat_hbm.at[i1_v], r1, sem)
            cp2 = pltpu.async_copy(feat_hbm.at[i2_v], r2, sem)
            cp0.wait()
            cp1.wait()
            cp2.wait()

            def group_body(g, pcarry):
                g16 = g * 16
                w0g = w0_v[pl.ds(g16, 16)]
                w1g = w1_v[pl.ds(g16, 16)]
                w2g = w2_v[pl.ds(g16, 16)]
                for j in range(16):
                    w0, w1, w2 = w0g[j], w1g[j], w2g[j]
                    pi = g16 + j
                    for db in range(d // 16):
                        ds = pl.ds(db * 16, 16)
                        r0[pi, ds] = (w0 * r0[pi, ds] + w1 * r1[pi, ds]
                                      + w2 * r2[pi, ds])
                return pcarry

            lax.fori_loop(0, c // 16, group_body, 0)
            pltpu.sync_copy(r0, out_hbm.at[sl])
            return carry

        lax.fori_loop(0, nchunk, chunk_body, 0)

    return sc_gather


def kernel(parent_coord, s_coord, s_feat, offset, new_offset):
    b = offset.shape[0]
    ns = s_coord.shape[0] // b
    nn = parent_coord.shape[0] // b
    d = s_feat.shape[1]
    tn = 256

    parent_coord_t = parent_coord.T

    bs = 1
    sc = _make_sc_gather(bs * nn, d, 128)
    outs = []
    for g in range(0, b, bs):
        i0, i1, i2, w0, w1, w2 = _topk_weights(
            parent_coord_t[:, g * nn:(g + bs) * nn],
            s_coord[g * ns:(g + bs) * ns],
            b=bs, nn=nn, ns=ns, tn=tn, b0=g)
        outs.append(sc(i0.reshape(-1), i1.reshape(-1), i2.reshape(-1),
                       w0.reshape(-1), w1.reshape(-1), w2.reshape(-1),
                       s_feat))
    return jnp.concatenate(outs, axis=0)
